# baseline (speedup 1.0000x reference)
import functools

import numpy as np
import jax
import jax.numpy as jnp
from jax import lax
from jax.experimental import pallas as pl
from jax.experimental.pallas import tpu as pltpu
from jax.experimental.pallas import tpu_sc as plsc

B, H, W, C = 16, 6, 512, 192
OVERLAP = 4
NUM_PVQS = 3
NUM_RVQS = 6
CODE_DIM = 256
CODE_SIZE = 8192
FIX_DIM = H * C
GROUP_DIM = FIX_DIM * OVERLAP // NUM_PVQS
T = W // OVERLAP
NTOK = B * T
TOK_TILE = 256
NT = NTOK // TOK_TILE
BT = TOK_TILE // T
HD = OVERLAP * C
KC = 2048
NKC = CODE_SIZE // KC

_NC, _NS = 2, 16
_NW = _NC * _NS
_TPW = NTOK // _NW

_o = np.arange(OVERLAP)[:, None]
_c = np.arange(C)[None, :]
_F = np.stack([( _o * FIX_DIM + _c * H + h).reshape(HD) for h in range(H)])
_G = _F // GROUP_DIM
_O = _F % GROUP_DIM
_GMASK = (_G[:, None, :] == np.arange(NUM_PVQS)[None, :, None]).astype(np.float32)


def _fold_weights(proj_down, proj_up):
    pdT = proj_down.transpose(0, 2, 1)
    wd = pdT[_G, _O]
    wd = wd[:, None] * _GMASK[..., None]
    wu = proj_up[_G, _O]
    wu = (wu[:, None] * _GMASK[..., None]).transpose(0, 1, 3, 2)
    return wd, wu


def _down_body(z_ref, wd_ref, zd_ref):
    h = pl.program_id(1)
    x = z_ref[...].reshape(TOK_TILE, HD)

    @pl.when(h == 0)
    def _():
        zd_ref[...] = jnp.zeros_like(zd_ref)

    for g in range(NUM_PVQS):
        zd_ref[g] += lax.dot_general(
            x, wd_ref[h, g], (((1,), (0,)), ((), ())),
            preferred_element_type=jnp.float32)


def _down(z_er, wd):
    return pl.pallas_call(
        _down_body,
        grid=(NT, H),
        in_specs=[
            pl.BlockSpec((BT, 1, T, HD), lambda t, h: (t, h, 0, 0)),
            pl.BlockSpec((H, NUM_PVQS, HD, CODE_DIM), lambda t, h: (0, 0, 0, 0)),
        ],
        out_specs=pl.BlockSpec((NUM_PVQS, TOK_TILE, CODE_DIM), lambda t, h: (0, t, 0)),
        out_shape=jax.ShapeDtypeStruct((NUM_PVQS, NTOK, CODE_DIM), jnp.float32),
    )(z_er, wd)


def _stream_body(s, resid_ref, emb_ref, codes_ref, ssq_ref):
    g = pl.program_id(0)
    t = pl.program_id(1)
    r = resid_ref[0]

    @pl.when(jnp.logical_and(g == 0, t == 0))
    def _():
        ssq_ref[...] = jnp.zeros_like(ssq_ref)

    ssq_ref[...] += jnp.broadcast_to(jnp.sum(r * r), (1, 128))

    zn = r * lax.rsqrt(jnp.sum(r * r, axis=-1, keepdims=True) + 1e-12)
    best_m = None
    best_i = None
    for k in range(NKC):
        emb_c = emb_ref[0, 0, pl.ds(k * KC, KC), :]
        sim = lax.dot_general(zn, emb_c, (((1,), (1,)), ((), ())),
                              preferred_element_type=jnp.float32)
        mk = jnp.max(sim, axis=-1, keepdims=True)
        iot = lax.broadcasted_iota(jnp.int32, sim.shape, 1) + k * KC
        ik = jnp.min(jnp.where(sim == mk, iot, CODE_SIZE), axis=-1, keepdims=True)
        if k == 0:
            best_m, best_i = mk, ik
        else:
            upd = mk > best_m
            best_i = jnp.where(upd, ik, best_i)
            best_m = jnp.maximum(best_m, mk)
    codes_ref[0, 0, 0] = best_i[:, 0].astype(jnp.int32)


def _stream(s, resid, emb_n):
    return pl.pallas_call(
        functools.partial(_stream_body, s),
        grid=(NUM_PVQS, NT),
        in_specs=[
            pl.BlockSpec((1, TOK_TILE, CODE_DIM), lambda g, t: (g, t, 0)),
            pl.BlockSpec((1, 1, CODE_SIZE, CODE_DIM), lambda g, t, s=s: (g, s, 0, 0)),
        ],
        out_specs=[
            pl.BlockSpec((1, 1, 1, TOK_TILE), lambda g, t: (g, t, 0, 0)),
            pl.BlockSpec((1, 128), lambda g, t: (0, 0)),
        ],
        out_shape=[
            jax.ShapeDtypeStruct((NUM_PVQS, NT, 1, TOK_TILE), jnp.int32),
            jax.ShapeDtypeStruct((1, 128), jnp.float32),
        ],
    )(resid, emb_n)


def _make_sc_update(stream_idx):
    mesh = plsc.VectorSubcoreMesh(core_axis_name="c", subcore_axis_name="s")

    @functools.partial(
        pl.kernel,
        mesh=mesh,
        out_type=jax.ShapeDtypeStruct((NUM_PVQS * NTOK, CODE_DIM), jnp.float32),
        scratch_types=[
            pltpu.VMEM((_TPW,), jnp.int32),
            pltpu.VMEM((_TPW, CODE_DIM), jnp.float32),
            pltpu.VMEM((_TPW, CODE_DIM), jnp.float32),
            pltpu.SemaphoreType.DMA,
        ],
    )
    def sc_update(codes_hbm, resid_hbm, table_hbm, out_hbm, idx_v, rows_v, r_v, sem):
        wid = lax.axis_index("s") * _NC + lax.axis_index("c")
        for g in range(NUM_PVQS):
            base = g * NTOK + wid * _TPW
            pltpu.sync_copy(codes_hbm.at[pl.ds(base, _TPW)], idx_v)
            off = jnp.int32((g * NUM_RVQS + stream_idx) * CODE_SIZE)
            for c in range(_TPW // 16):
                sl = pl.ds(c * 16, 16)
                idx_v[sl] = idx_v[sl] + off
            pltpu.async_copy(table_hbm.at[idx_v], rows_v, sem).wait()
            pltpu.sync_copy(resid_hbm.at[pl.ds(base, _TPW)], r_v)

            def body(i, carry):
                for c in range(CODE_DIM // 16):
                    sl = (i, pl.ds(c * 16, 16))
                    r_v[sl] = r_v[sl] - rows_v[sl]
                return carry

            lax.fori_loop(0, _TPW, body, 0)
            pltpu.sync_copy(r_v, out_hbm.at[pl.ds(base, _TPW)])

    return sc_update


def _up_body(zd_ref, r_ref, wu_ref, zq_ref, ssq_ref):
    t = pl.program_id(0)
    h = pl.program_id(1)

    @pl.when(jnp.logical_and(t == 0, h == 0))
    def _():
        ssq_ref[...] = jnp.zeros_like(ssq_ref)

    @pl.when(h == 0)
    def _():
        r = r_ref[...]
        ssq_ref[...] += jnp.broadcast_to(jnp.sum(r * r), (1, 128))

    acc = None
    for g in range(NUM_PVQS):
        part = lax.dot_general(
            zd_ref[g] - r_ref[g], wu_ref[h, g], (((1,), (0,)), ((), ())),
            preferred_element_type=jnp.float32)
        acc = part if acc is None else acc + part
    zq_ref[...] = acc.reshape(BT, 1, T, HD)


def _up(zd, resid, wu):
    return pl.pallas_call(
        _up_body,
        grid=(NT, H),
        in_specs=[
            pl.BlockSpec((NUM_PVQS, TOK_TILE, CODE_DIM), lambda t, h: (0, t, 0)),
            pl.BlockSpec((NUM_PVQS, TOK_TILE, CODE_DIM), lambda t, h: (0, t, 0)),
            pl.BlockSpec((H, NUM_PVQS, CODE_DIM, HD), lambda t, h: (0, 0, 0, 0)),
        ],
        out_specs=[
            pl.BlockSpec((BT, 1, T, HD), lambda t, h: (t, h, 0, 0)),
            pl.BlockSpec((1, 128), lambda t, h: (0, 0)),
        ],
        out_shape=[
            jax.ShapeDtypeStruct((B, H, T, HD), jnp.float32),
            jax.ShapeDtypeStruct((1, 128), jnp.float32),
        ],
    )(zd, resid, wu)


def kernel(z_e, num_streams, proj_down, proj_up, codebooks):
    b = z_e.shape[0]
    z_er = z_e.reshape(b, H, T, HD)
    wd, wu = _fold_weights(proj_down, proj_up)

    emb_n = codebooks * lax.rsqrt(
        jnp.sum(codebooks * codebooks, axis=-1, keepdims=True) + 1e-12)
    table = emb_n.reshape(NUM_PVQS * NUM_RVQS * CODE_SIZE, CODE_DIM)

    zd = _down(z_er, wd)
    resid = zd
    codes_list = []
    ssq_list = []
    for s in range(NUM_RVQS):
        codes4, ssq = _stream(s, resid, emb_n)
        codes = codes4.reshape(NUM_PVQS, NTOK)
        ssq_list.append(ssq[0, 0])
        resid_flat = _make_sc_update(s)(
            codes.reshape(NUM_PVQS * NTOK),
            resid.reshape(NUM_PVQS * NTOK, CODE_DIM),
            table)
        resid = resid_flat.reshape(NUM_PVQS, NTOK, CODE_DIM)
        codes_list.append(codes)

    zqr, ssq6 = _up(zd, resid, wu)

    denom = jnp.float32(NUM_PVQS * NTOK * CODE_DIM)
    cm = (sum(ssq_list[1:]) + ssq6[0, 0]) / denom
    cb = cm

    codes_all = jnp.stack(codes_list, axis=0).reshape(NUM_RVQS, NUM_PVQS, b, T)
    indices = codes_all.transpose(2, 0, 1, 3)

    z_q = zqr.reshape(b, H, W, C).reshape(b, H * W, C)
    return z_q, indices, cm, cb


# speedup vs baseline: 1.0365x; 1.0365x over previous
import functools

import numpy as np
import jax
import jax.numpy as jnp
from jax import lax
from jax.experimental import pallas as pl
from jax.experimental.pallas import tpu as pltpu
from jax.experimental.pallas import tpu_sc as plsc

B, H, W, C = 16, 6, 512, 192
OVERLAP = 4
NUM_PVQS = 3
NUM_RVQS = 6
CODE_DIM = 256
CODE_SIZE = 8192
FIX_DIM = H * C
GROUP_DIM = FIX_DIM * OVERLAP // NUM_PVQS
T = W // OVERLAP
NTOK = B * T
TOK_TILE = 256
NT = NTOK // TOK_TILE
BT = TOK_TILE // T
HD = OVERLAP * C
KC = 2048
NKC = CODE_SIZE // KC

_NC, _NS = 2, 16
_NW = _NC * _NS
_TPW = NTOK // _NW

_o = np.arange(OVERLAP)[:, None]
_c = np.arange(C)[None, :]
_F = np.stack([( _o * FIX_DIM + _c * H + h).reshape(HD) for h in range(H)])
_G = _F // GROUP_DIM
_O = _F % GROUP_DIM
_GMASK = (_G[:, None, :] == np.arange(NUM_PVQS)[None, :, None]).astype(np.float32)


def _fold_weights(proj_down, proj_up):
    pdT = proj_down.transpose(0, 2, 1)
    wd = pdT[_G, _O]
    wd = wd[:, None] * _GMASK[..., None]
    wu = proj_up[_G, _O]
    wu = (wu[:, None] * _GMASK[..., None]).transpose(0, 1, 3, 2)
    return wd, wu


def _down_body(z_ref, wd_ref, zd_ref):
    h = pl.program_id(1)
    x = z_ref[...].reshape(TOK_TILE, HD)

    @pl.when(h == 0)
    def _():
        zd_ref[...] = jnp.zeros_like(zd_ref)

    for g in range(NUM_PVQS):
        zd_ref[g] += lax.dot_general(
            x, wd_ref[h, g], (((1,), (0,)), ((), ())),
            preferred_element_type=jnp.float32)


def _down(z_er, wd):
    return pl.pallas_call(
        _down_body,
        grid=(NT, H),
        in_specs=[
            pl.BlockSpec((BT, 1, T, HD), lambda t, h: (t, h, 0, 0)),
            pl.BlockSpec((H, NUM_PVQS, HD, CODE_DIM), lambda t, h: (0, 0, 0, 0)),
        ],
        out_specs=pl.BlockSpec((NUM_PVQS, TOK_TILE, CODE_DIM), lambda t, h: (0, t, 0)),
        out_shape=jax.ShapeDtypeStruct((NUM_PVQS, NTOK, CODE_DIM), jnp.float32),
    )(z_er, wd)


def _stream_body(resid_ref, emb_ref, codes_ref, ssq_ref):
    t = pl.program_id(0)
    r = resid_ref[...]

    @pl.when(t == 0)
    def _():
        ssq_ref[...] = jnp.zeros_like(ssq_ref)

    ssq_ref[...] += jnp.broadcast_to(jnp.sum(r * r), (1, 128))

    zn = r * lax.rsqrt(jnp.sum(r * r, axis=-1, keepdims=True) + 1e-12)
    best_m = None
    best_i = None
    for k in range(NKC):
        emb_c = emb_ref[0, 0, pl.ds(k * KC, KC), :]
        sim = lax.dot_general(zn, emb_c, (((1,), (1,)), ((), ())),
                              preferred_element_type=jnp.float32)
        mk = jnp.max(sim, axis=-1, keepdims=True)
        iot = lax.broadcasted_iota(jnp.int32, sim.shape, 1) + k * KC
        ik = jnp.min(jnp.where(sim == mk, iot, CODE_SIZE), axis=-1, keepdims=True)
        if k == 0:
            best_m, best_i = mk, ik
        else:
            upd = mk > best_m
            best_i = jnp.where(upd, ik, best_i)
            best_m = jnp.maximum(best_m, mk)
    codes_ref[0, 0] = best_i[:, 0].astype(jnp.int32)


def _stream(s, g, resid_g, emb_n):
    return pl.pallas_call(
        _stream_body,
        grid=(NT,),
        in_specs=[
            pl.BlockSpec((TOK_TILE, CODE_DIM), lambda t: (t, 0)),
            pl.BlockSpec((1, 1, CODE_SIZE, CODE_DIM),
                         lambda t, g=g, s=s: (g, s, 0, 0)),
        ],
        out_specs=[
            pl.BlockSpec((1, 1, TOK_TILE), lambda t: (t, 0, 0)),
            pl.BlockSpec((1, 128), lambda t: (0, 0)),
        ],
        out_shape=[
            jax.ShapeDtypeStruct((NT, 1, TOK_TILE), jnp.int32),
            jax.ShapeDtypeStruct((1, 128), jnp.float32),
        ],
    )(resid_g, emb_n)


def _make_sc_update(stream_idx, group_idx):
    mesh = plsc.VectorSubcoreMesh(core_axis_name="c", subcore_axis_name="s")

    @functools.partial(
        pl.kernel,
        mesh=mesh,
        out_type=jax.ShapeDtypeStruct((NTOK, CODE_DIM), jnp.float32),
        scratch_types=[
            pltpu.VMEM((_TPW,), jnp.int32),
            pltpu.VMEM((_TPW, CODE_DIM), jnp.float32),
            pltpu.VMEM((_TPW, CODE_DIM), jnp.float32),
            pltpu.SemaphoreType.DMA,
        ],
    )
    def sc_update(codes_hbm, resid_hbm, table_hbm, out_hbm, idx_v, rows_v, r_v, sem):
        wid = lax.axis_index("s") * _NC + lax.axis_index("c")
        if True:
            base = wid * _TPW
            pltpu.sync_copy(codes_hbm.at[pl.ds(base, _TPW)], idx_v)
            off = jnp.int32((group_idx * NUM_RVQS + stream_idx) * CODE_SIZE)
            for c in range(_TPW // 16):
                sl = pl.ds(c * 16, 16)
                idx_v[sl] = idx_v[sl] + off
            pltpu.async_copy(table_hbm.at[idx_v], rows_v, sem).wait()
            pltpu.sync_copy(resid_hbm.at[pl.ds(base, _TPW)], r_v)

            def body(i, carry):
                for c in range(CODE_DIM // 16):
                    sl = (i, pl.ds(c * 16, 16))
                    r_v[sl] = r_v[sl] - rows_v[sl]
                return carry

            lax.fori_loop(0, _TPW, body, 0)
            pltpu.sync_copy(r_v, out_hbm.at[pl.ds(base, _TPW)])

    return sc_update


def _up_body(zd_ref, r0_ref, r1_ref, r2_ref, wu_ref, zq_ref, ssq_ref):
    t = pl.program_id(0)
    h = pl.program_id(1)
    r_refs = (r0_ref, r1_ref, r2_ref)

    @pl.when(jnp.logical_and(t == 0, h == 0))
    def _():
        ssq_ref[...] = jnp.zeros_like(ssq_ref)

    @pl.when(h == 0)
    def _():
        s = None
        for rr in r_refs:
            r = rr[...]
            s = jnp.sum(r * r) if s is None else s + jnp.sum(r * r)
        ssq_ref[...] += jnp.broadcast_to(s, (1, 128))

    acc = None
    for g in range(NUM_PVQS):
        part = lax.dot_general(
            zd_ref[g] - r_refs[g][...], wu_ref[h, g], (((1,), (0,)), ((), ())),
            preferred_element_type=jnp.float32)
        acc = part if acc is None else acc + part
    zq_ref[...] = acc.reshape(BT, 1, T, HD)


def _up(zd, resid, wu):
    rspec = pl.BlockSpec((TOK_TILE, CODE_DIM), lambda t, h: (t, 0))
    return pl.pallas_call(
        _up_body,
        grid=(NT, H),
        in_specs=[
            pl.BlockSpec((NUM_PVQS, TOK_TILE, CODE_DIM), lambda t, h: (0, t, 0)),
            rspec, rspec, rspec,
            pl.BlockSpec((H, NUM_PVQS, CODE_DIM, HD), lambda t, h: (0, 0, 0, 0)),
        ],
        out_specs=[
            pl.BlockSpec((BT, 1, T, HD), lambda t, h: (t, h, 0, 0)),
            pl.BlockSpec((1, 128), lambda t, h: (0, 0)),
        ],
        out_shape=[
            jax.ShapeDtypeStruct((B, H, T, HD), jnp.float32),
            jax.ShapeDtypeStruct((1, 128), jnp.float32),
        ],
    )(zd, resid[0], resid[1], resid[2], wu)


def kernel(z_e, num_streams, proj_down, proj_up, codebooks):
    b = z_e.shape[0]
    z_er = z_e.reshape(b, H, T, HD)
    wd, wu = _fold_weights(proj_down, proj_up)

    emb_n = codebooks * lax.rsqrt(
        jnp.sum(codebooks * codebooks, axis=-1, keepdims=True) + 1e-12)
    table = emb_n.reshape(NUM_PVQS * NUM_RVQS * CODE_SIZE, CODE_DIM)

    zd = _down(z_er, wd)
    resid = [zd[g] for g in range(NUM_PVQS)]
    codes_list = []
    ssq_list = []
    for s in range(NUM_RVQS):
        codes_s = []
        ssq_s = []
        for g in range(NUM_PVQS):
            codes3, ssq = _stream(s, g, resid[g], emb_n)
            codes_s.append(codes3.reshape(NTOK))
            ssq_s.append(ssq[0, 0])
        for g in range(NUM_PVQS):
            resid[g] = _make_sc_update(s, g)(codes_s[g], resid[g], table)
        ssq_list.append(sum(ssq_s))
        codes_list.append(jnp.stack(codes_s, axis=0))

    zqr, ssq6 = _up(zd, resid, wu)

    denom = jnp.float32(NUM_PVQS * NTOK * CODE_DIM)
    cm = (sum(ssq_list[1:]) + ssq6[0, 0]) / denom
    cb = cm

    codes_all = jnp.stack(codes_list, axis=0).reshape(NUM_RVQS, NUM_PVQS, b, T)
    indices = codes_all.transpose(2, 0, 1, 3)

    z_q = zqr.reshape(b, H, W, C).reshape(b, H * W, C)
    return z_q, indices, cm, cb


# speedup vs baseline: 1.1082x; 1.0692x over previous
import functools

import numpy as np
import jax
import jax.numpy as jnp
from jax import lax
from jax.experimental import pallas as pl
from jax.experimental.pallas import tpu as pltpu
from jax.experimental.pallas import tpu_sc as plsc

B, H, W, C = 16, 6, 512, 192
OVERLAP = 4
NUM_PVQS = 3
NUM_RVQS = 6
CODE_DIM = 256
CODE_SIZE = 8192
FIX_DIM = H * C
GROUP_DIM = FIX_DIM * OVERLAP // NUM_PVQS
T = W // OVERLAP
NTOK = B * T
TOK_TILE = 512
NT = NTOK // TOK_TILE
BT = TOK_TILE // T
HD = OVERLAP * C
KC = 2048
NKC = CODE_SIZE // KC

_NC, _NS = 2, 16
_NW = _NC * _NS
_TPW = NTOK // _NW

_o = np.arange(OVERLAP)[:, None]
_c = np.arange(C)[None, :]
_F = np.stack([( _o * FIX_DIM + _c * H + h).reshape(HD) for h in range(H)])
_G = _F // GROUP_DIM
_O = _F % GROUP_DIM
_GMASK = (_G[:, None, :] == np.arange(NUM_PVQS)[None, :, None]).astype(np.float32)


def _fold_weights(proj_down, proj_up):
    pdT = proj_down.transpose(0, 2, 1)
    wd = pdT[_G, _O]
    wd = wd[:, None] * _GMASK[..., None]
    wu = proj_up[_G, _O]
    wu = (wu[:, None] * _GMASK[..., None]).transpose(0, 1, 3, 2)
    return wd, wu


def _down_body(z_ref, wd_ref, zd_ref):
    h = pl.program_id(1)
    x = z_ref[...].reshape(TOK_TILE, HD)

    @pl.when(h == 0)
    def _():
        zd_ref[...] = jnp.zeros_like(zd_ref)

    for g in range(NUM_PVQS):
        zd_ref[g] += lax.dot_general(
            x, wd_ref[h, g], (((1,), (0,)), ((), ())),
            preferred_element_type=jnp.float32)


def _down(z_er, wd):
    return pl.pallas_call(
        _down_body,
        grid=(NT, H),
        in_specs=[
            pl.BlockSpec((BT, 1, T, HD), lambda t, h: (t, h, 0, 0)),
            pl.BlockSpec((H, NUM_PVQS, HD, CODE_DIM), lambda t, h: (0, 0, 0, 0)),
        ],
        out_specs=pl.BlockSpec((NUM_PVQS, TOK_TILE, CODE_DIM), lambda t, h: (0, t, 0)),
        out_shape=jax.ShapeDtypeStruct((NUM_PVQS, NTOK, CODE_DIM), jnp.float32),
    )(z_er, wd)


def _stream_body(resid_ref, emb_ref, codes_ref, ssq_ref):
    t = pl.program_id(0)
    r = resid_ref[...]

    @pl.when(t == 0)
    def _():
        ssq_ref[...] = jnp.zeros_like(ssq_ref)

    ssq_ref[...] += jnp.broadcast_to(jnp.sum(r * r), (1, 128))

    zn = r * lax.rsqrt(jnp.sum(r * r, axis=-1, keepdims=True) + 1e-12)
    iot = lax.broadcasted_iota(jnp.int32, (TOK_TILE, KC), 1)
    best_m = None
    best_i = None
    for k in range(NKC):
        emb_c = emb_ref[0, 0, pl.ds(k * KC, KC), :]
        sim = lax.dot_general(zn, emb_c, (((1,), (1,)), ((), ())),
                              preferred_element_type=jnp.float32)
        mk = jnp.max(sim, axis=-1, keepdims=True)
        ik = jnp.min(jnp.where(sim == mk, iot, KC), axis=-1, keepdims=True) + k * KC
        if k == 0:
            best_m, best_i = mk, ik
        else:
            upd = mk > best_m
            best_i = jnp.where(upd, ik, best_i)
            best_m = jnp.maximum(best_m, mk)
    codes_ref[0, 0] = best_i[:, 0].astype(jnp.int32)


def _stream(s, g, resid_g, emb_n):
    return pl.pallas_call(
        _stream_body,
        grid=(NT,),
        in_specs=[
            pl.BlockSpec((TOK_TILE, CODE_DIM), lambda t: (t, 0)),
            pl.BlockSpec((1, 1, CODE_SIZE, CODE_DIM),
                         lambda t, g=g, s=s: (g, s, 0, 0)),
        ],
        out_specs=[
            pl.BlockSpec((1, 1, TOK_TILE), lambda t: (t, 0, 0)),
            pl.BlockSpec((1, 128), lambda t: (0, 0)),
        ],
        out_shape=[
            jax.ShapeDtypeStruct((NT, 1, TOK_TILE), jnp.int32),
            jax.ShapeDtypeStruct((1, 128), jnp.float32),
        ],
    )(resid_g, emb_n)


def _make_sc_update(stream_idx, group_idx):
    mesh = plsc.VectorSubcoreMesh(core_axis_name="c", subcore_axis_name="s")

    @functools.partial(
        pl.kernel,
        mesh=mesh,
        out_type=jax.ShapeDtypeStruct((NTOK, CODE_DIM), jnp.float32),
        scratch_types=[
            pltpu.VMEM((_TPW,), jnp.int32),
            pltpu.VMEM((_TPW, CODE_DIM), jnp.float32),
            pltpu.VMEM((_TPW, CODE_DIM), jnp.float32),
            pltpu.SemaphoreType.DMA,
        ],
    )
    def sc_update(codes_hbm, resid_hbm, table_hbm, out_hbm, idx_v, rows_v, r_v, sem):
        wid = lax.axis_index("s") * _NC + lax.axis_index("c")
        if True:
            base = wid * _TPW
            pltpu.sync_copy(codes_hbm.at[pl.ds(base, _TPW)], idx_v)
            off = jnp.int32((group_idx * NUM_RVQS + stream_idx) * CODE_SIZE)
            for c in range(_TPW // 16):
                sl = pl.ds(c * 16, 16)
                idx_v[sl] = idx_v[sl] + off
            pltpu.async_copy(table_hbm.at[idx_v], rows_v, sem).wait()
            pltpu.sync_copy(resid_hbm.at[pl.ds(base, _TPW)], r_v)

            def body(i, carry):
                for c in range(CODE_DIM // 16):
                    sl = (i, pl.ds(c * 16, 16))
                    r_v[sl] = r_v[sl] - rows_v[sl]
                return carry

            lax.fori_loop(0, _TPW, body, 0)
            pltpu.sync_copy(r_v, out_hbm.at[pl.ds(base, _TPW)])

    return sc_update


def _up_body(zd_ref, r0_ref, r1_ref, r2_ref, wu_ref, zq_ref, ssq_ref):
    t = pl.program_id(0)
    h = pl.program_id(1)
    r_refs = (r0_ref, r1_ref, r2_ref)

    @pl.when(jnp.logical_and(t == 0, h == 0))
    def _():
        ssq_ref[...] = jnp.zeros_like(ssq_ref)

    @pl.when(h == 0)
    def _():
        s = None
        for rr in r_refs:
            r = rr[...]
            s = jnp.sum(r * r) if s is None else s + jnp.sum(r * r)
        ssq_ref[...] += jnp.broadcast_to(s, (1, 128))

    acc = None
    for g in range(NUM_PVQS):
        part = lax.dot_general(
            zd_ref[g] - r_refs[g][...], wu_ref[h, g], (((1,), (0,)), ((), ())),
            preferred_element_type=jnp.float32)
        acc = part if acc is None else acc + part
    zq_ref[...] = acc.reshape(BT, 1, T, HD)


def _up(zd, resid, wu):
    rspec = pl.BlockSpec((TOK_TILE, CODE_DIM), lambda t, h: (t, 0))
    return pl.pallas_call(
        _up_body,
        grid=(NT, H),
        in_specs=[
            pl.BlockSpec((NUM_PVQS, TOK_TILE, CODE_DIM), lambda t, h: (0, t, 0)),
            rspec, rspec, rspec,
            pl.BlockSpec((H, NUM_PVQS, CODE_DIM, HD), lambda t, h: (0, 0, 0, 0)),
        ],
        out_specs=[
            pl.BlockSpec((BT, 1, T, HD), lambda t, h: (t, h, 0, 0)),
            pl.BlockSpec((1, 128), lambda t, h: (0, 0)),
        ],
        out_shape=[
            jax.ShapeDtypeStruct((B, H, T, HD), jnp.float32),
            jax.ShapeDtypeStruct((1, 128), jnp.float32),
        ],
    )(zd, resid[0], resid[1], resid[2], wu)


def kernel(z_e, num_streams, proj_down, proj_up, codebooks):
    b = z_e.shape[0]
    z_er = z_e.reshape(b, H, T, HD)
    wd, wu = _fold_weights(proj_down, proj_up)

    emb_n = codebooks * lax.rsqrt(
        jnp.sum(codebooks * codebooks, axis=-1, keepdims=True) + 1e-12)
    table = emb_n.reshape(NUM_PVQS * NUM_RVQS * CODE_SIZE, CODE_DIM)

    zd = _down(z_er, wd)
    resid = [zd[g] for g in range(NUM_PVQS)]
    codes_list = []
    ssq_list = []
    for s in range(NUM_RVQS):
        codes_s = []
        ssq_s = []
        for g in range(NUM_PVQS):
            codes3, ssq = _stream(s, g, resid[g], emb_n)
            codes_s.append(codes3.reshape(NTOK))
            ssq_s.append(ssq[0, 0])
        for g in range(NUM_PVQS):
            resid[g] = _make_sc_update(s, g)(codes_s[g], resid[g], table)
        ssq_list.append(sum(ssq_s))
        codes_list.append(jnp.stack(codes_s, axis=0))

    zqr, ssq6 = _up(zd, resid, wu)

    denom = jnp.float32(NUM_PVQS * NTOK * CODE_DIM)
    cm = (sum(ssq_list[1:]) + ssq6[0, 0]) / denom
    cb = cm

    codes_all = jnp.stack(codes_list, axis=0).reshape(NUM_RVQS, NUM_PVQS, b, T)
    indices = codes_all.transpose(2, 0, 1, 3)

    z_q = zqr.reshape(b, H, W, C).reshape(b, H * W, C)
    return z_q, indices, cm, cb
